# baseline (speedup 1.0000x reference)
import functools

import jax
import jax.numpy as jnp
from jax import lax
from jax.experimental import pallas as pl
from jax.experimental.pallas import tpu as pltpu
from jax.experimental.pallas import tpu_sc as plsc

_LANES = 16
_NUM_CORES = 2
_NUM_SUBCORES = 16


@functools.lru_cache(maxsize=None)
def _build_sc_pool(B, S, D):
    NW = _NUM_CORES * _NUM_SUBCORES
    BPW = B // NW
    K = D // _LANES
    chunks = []
    o = 0
    while o < S:
        n = min(128, S - o)
        chunks.append((o, n))
        o += n

    mesh = plsc.VectorSubcoreMesh(
        core_axis_name="c", subcore_axis_name="s",
        num_cores=_NUM_CORES, num_subcores=_NUM_SUBCORES)

    @functools.partial(
        pl.kernel,
        out_type=jax.ShapeDtypeStruct((B, D), jnp.float32),
        mesh=mesh,
        scratch_types=[
            pltpu.VMEM((BPW * S,), jnp.int32),
            pltpu.VMEM((S, D), jnp.float32),
            pltpu.VMEM((S, D), jnp.float32),
            pltpu.VMEM((BPW, D), jnp.float32),
            pltpu.SemaphoreType.DMA,
            pltpu.SemaphoreType.DMA,
        ],
        compiler_params=pltpu.CompilerParams(use_tc_tiling_on_sc=False),
    )
    def sc_pool(idx_hbm, table_hbm, out_hbm, idx_v, buf0, buf1, pooled_v,
                sem0, sem1):
        wid = lax.axis_index("s") * _NUM_CORES + lax.axis_index("c")
        pltpu.sync_copy(idx_hbm.at[wid], idx_v)
        bufs = (buf0, buf1)
        sems = (sem0, sem1)

        def issue(b, buf, sem):
            off = b * S
            for (o, n) in chunks:
                pltpu.async_copy(
                    table_hbm.at[idx_v.at[pl.ds(off + o, n)]],
                    buf.at[pl.ds(o, n)],
                    sem,
                )

        def wait(buf, sem):
            pltpu.make_async_copy(table_hbm.at[pl.ds(0, S)], buf, sem).wait()

        inv = jnp.float32(1.0 / S)

        def accumulate(buf, b):
            zero = jnp.zeros((_LANES,), jnp.float32)

            def rbody(r, accs):
                a = list(accs)
                base = r * 4
                for u in range(4):
                    for k2 in range(K):
                        j = (u % 2) * K + k2
                        a[j] = a[j] + buf[base + u, pl.ds(k2 * _LANES, _LANES)]
                return tuple(a)

            accs = lax.fori_loop(0, S // 4, rbody, (zero,) * (2 * K))
            for k2 in range(K):
                pooled_v[b, pl.ds(k2 * _LANES, _LANES)] = (
                    accs[k2] + accs[K + k2]) * inv

        issue(0, buf0, sem0)
        issue(1, buf1, sem1)

        def outer(g, carry):
            for t in range(2):
                b = g * 2 + t
                wait(bufs[t], sems[t])
                accumulate(bufs[t], b)

                @pl.when(b + 2 < BPW)
                def _():
                    issue(b + 2, bufs[t], sems[t])
            return carry

        lax.fori_loop(0, BPW // 2, outer, 0)
        pltpu.sync_copy(pooled_v, out_hbm.at[pl.ds(wid * BPW, BPW)])

    return sc_pool


def _tc_linear(pooled, fc_w, fc_b):
    B, D = pooled.shape
    O = fc_w.shape[1]
    BB = 512

    def body(x_ref, w_ref, b_ref, o_ref):
        o_ref[...] = (
            jnp.dot(x_ref[...], w_ref[...], preferred_element_type=jnp.float32)
            + b_ref[...]
        )

    return pl.pallas_call(
        body,
        out_shape=jax.ShapeDtypeStruct((B, O), jnp.float32),
        grid=(B // BB,),
        in_specs=[
            pl.BlockSpec((BB, D), lambda i: (i, 0)),
            pl.BlockSpec((D, O), lambda i: (0, 0)),
            pl.BlockSpec((1, O), lambda i: (0, 0)),
        ],
        out_specs=pl.BlockSpec((BB, O), lambda i: (i, 0)),
    )(pooled, fc_w, fc_b.reshape(1, O))


def kernel(input_ids, embedding, fc_w, fc_b):
    B, S = input_ids.shape
    _, D = embedding.shape
    NW = _NUM_CORES * _NUM_SUBCORES
    idx = input_ids.astype(jnp.int32).reshape(NW, (B // NW) * S)
    pooled = _build_sc_pool(B, S, D)(idx, embedding)
    return _tc_linear(pooled, fc_w, fc_b)
